# baseline (speedup 1.0000x reference)
import functools

import jax
import jax.numpy as jnp
from jax import lax
from jax.experimental import pallas as pl
from jax.experimental.pallas import tpu as pltpu
from jax.experimental.pallas import tpu_sc as plsc

_D = 128
_NC = 2
_NS = 16
_NW = _NC * _NS
_CHUNK = 128


def _make_gather(batch: int):
  b_per_w = batch // _NW
  n_chunks = b_per_w // _CHUNK
  mesh = plsc.VectorSubcoreMesh(core_axis_name="c", subcore_axis_name="s")

  @functools.partial(
      pl.kernel,
      mesh=mesh,
      out_type=jax.ShapeDtypeStruct((batch, _D), jnp.float32),
      scratch_types=[
          pltpu.VMEM((n_chunks, _CHUNK), jnp.int32),
          pltpu.VMEM((b_per_w, _D), jnp.float32),
          [pltpu.SemaphoreType.DMA] * n_chunks,
          pltpu.SemaphoreType.DMA,
      ],
  )
  def gather_kernel(table_hbm, idx_hbm, out_hbm, idx_v, rows_v, gsems, wsem):
    wid = lax.axis_index("s") * _NC + lax.axis_index("c")
    base = wid * b_per_w
    pltpu.sync_copy(idx_hbm.at[wid], idx_v)
    gathers = [
        pltpu.async_copy(
            table_hbm.at[idx_v.at[j]],
            rows_v.at[pl.ds(j * _CHUNK, _CHUNK)],
            gsems[j],
        )
        for j in range(n_chunks)
    ]
    writes = []
    for j in range(n_chunks):
      gathers[j].wait()
      writes.append(
          pltpu.async_copy(
              rows_v.at[pl.ds(j * _CHUNK, _CHUNK)],
              out_hbm.at[pl.ds(base + j * _CHUNK, _CHUNK)],
              wsem,
          )
      )
    for w in writes:
      w.wait()

  return gather_kernel


@jax.jit
def kernel(features, nodes):
  batch = nodes.shape[0]
  idx = nodes.astype(jnp.int32).reshape(_NW, batch // (_NW * _CHUNK), _CHUNK)
  return _make_gather(batch)(features, idx)


# speedup vs baseline: 1.0053x; 1.0053x over previous
import functools

import jax
import jax.numpy as jnp
from jax import lax
from jax.experimental import pallas as pl
from jax.experimental.pallas import tpu as pltpu
from jax.experimental.pallas import tpu_sc as plsc

_D = 128
_NC = 2
_NS = 16
_NW = _NC * _NS
_CHUNK = 128


def _make_gather(batch: int):
  b_per_w = batch // _NW
  n_chunks = b_per_w // _CHUNK
  half = n_chunks // 2
  mesh = plsc.VectorSubcoreMesh(core_axis_name="c", subcore_axis_name="s")

  @functools.partial(
      pl.kernel,
      mesh=mesh,
      out_type=jax.ShapeDtypeStruct((batch, _D), jnp.float32),
      scratch_types=[
          pltpu.VMEM((b_per_w,), jnp.int32),
          pltpu.VMEM((b_per_w, _D), jnp.float32),
          [pltpu.SemaphoreType.DMA] * n_chunks,
          pltpu.SemaphoreType.DMA,
      ],
  )
  def gather_kernel(table_hbm, idx_hbm, out_hbm, idx_v, rows_v, gsems, wsem):
    wid = lax.axis_index("s") * _NC + lax.axis_index("c")
    base = wid * b_per_w
    pltpu.sync_copy(idx_hbm.at[pl.ds(base, b_per_w)], idx_v)
    gathers = [
        pltpu.async_copy(
            table_hbm.at[idx_v.at[pl.ds(j * _CHUNK, _CHUNK)]],
            rows_v.at[pl.ds(j * _CHUNK, _CHUNK)],
            gsems[j],
        )
        for j in range(n_chunks)
    ]
    writes = []
    for h in range(2):
      for j in range(h * half, (h + 1) * half):
        gathers[j].wait()
      writes.append(
          pltpu.async_copy(
              rows_v.at[pl.ds(h * half * _CHUNK, half * _CHUNK)],
              out_hbm.at[pl.ds(base + h * half * _CHUNK, half * _CHUNK)],
              wsem,
          )
      )
    for w in writes:
      w.wait()

  return gather_kernel


@jax.jit
def kernel(features, nodes):
  return _make_gather(nodes.shape[0])(features, nodes.astype(jnp.int32))


# speedup vs baseline: 1.0197x; 1.0143x over previous
import functools

import jax
import jax.numpy as jnp
from jax import lax
from jax.experimental import pallas as pl
from jax.experimental.pallas import tpu as pltpu
from jax.experimental.pallas import tpu_sc as plsc

_D = 128
_NC = 2
_NS = 16
_NW = _NC * _NS
_CHUNK = 128


def _make_gather(batch: int):
  b_per_w = batch // _NW
  n_chunks = b_per_w // _CHUNK
  mesh = plsc.VectorSubcoreMesh(core_axis_name="c", subcore_axis_name="s")

  @functools.partial(
      pl.kernel,
      mesh=mesh,
      out_type=jax.ShapeDtypeStruct((batch, _D), jnp.float32),
      scratch_types=[
          pltpu.VMEM((b_per_w,), jnp.int32),
          pltpu.VMEM((b_per_w, _D), jnp.float32),
          pltpu.SemaphoreType.DMA,
      ],
  )
  def gather_kernel(table_hbm, idx_hbm, out_hbm, idx_v, rows_v, sem):
    wid = lax.axis_index("s") * _NC + lax.axis_index("c")
    base = wid * b_per_w
    pltpu.sync_copy(idx_hbm.at[pl.ds(base, b_per_w)], idx_v)
    gathers = [
        pltpu.async_copy(
            table_hbm.at[idx_v.at[pl.ds(j * _CHUNK, _CHUNK)]],
            rows_v.at[pl.ds(j * _CHUNK, _CHUNK)],
            sem,
        )
        for j in range(n_chunks)
    ]
    for g in gathers:
      g.wait()
    pltpu.sync_copy(rows_v, out_hbm.at[pl.ds(base, b_per_w)])

  return gather_kernel


@jax.jit
def kernel(features, nodes):
  return _make_gather(nodes.shape[0])(features, nodes.astype(jnp.int32))
